# Initial kernel scaffold; baseline (speedup 1.0000x reference)
import jax
import jax.numpy as jnp
from jax.experimental import pallas as pl


def kernel(x, edge_index, batch, W1, b1, W2, b2, W3, b3, W4, b4, Wm, bm, Wm2, bm2, Wl, bl):
    raise NotImplementedError("write your pallas kernel here")


# speedup vs baseline: 5.4723x; 5.4723x over previous
import functools

import jax
import jax.numpy as jnp
from jax import lax
from jax.experimental import pallas as pl
from jax.experimental.pallas import tpu as pltpu
from jax.experimental.pallas import tpu_sc as plsc

_N = 10000
_E = 160000
_D = 256
_G = 64
_HW = 128
_K = 128
_NP = 10240
_EP = 163840
_EPT = _EP // 16
_CPT = _EPT // _K
_RPT = _NP // 16
_RB = 400

_mesh = plsc.VectorSubcoreMesh(
    core_axis_name="c", subcore_axis_name="s", num_cores=2, num_subcores=16)

_f32 = jnp.float32


@functools.partial(
    pl.kernel,
    out_type=(jax.ShapeDtypeStruct((_NP, _HW), _f32),
              jax.ShapeDtypeStruct((_NP, _HW), _f32)),
    mesh=_mesh,
    scratch_types=[
        pltpu.VMEM((_CPT // 2, _K), jnp.int32),
        pltpu.VMEM((_CPT // 2, _K), jnp.int32),
        pltpu.VMEM((_K, _HW), _f32),
        pltpu.VMEM((_K, _HW), _f32),
        pltpu.VMEM_SHARED((_NP, _HW), _f32),
        pltpu.SemaphoreType.DMA,
        pltpu.SemaphoreType.DMA,
    ],
)
def _spmv(qa_hbm, qb_hbm, srcp_hbm, dstp_hbm, z128_hbm, sa_hbm, sb_hbm,
          sidx2, didx2, r0, r1, acc, sem0, sem1):
  c = lax.axis_index("c")
  s = lax.axis_index("s")
  hc = _CPT // 2
  pltpu.sync_copy(z128_hbm, acc.at[pl.ds(s * _RPT, _RPT)])
  plsc.subcore_barrier()

  def run_half(q_hbm):
    for p in (0, 1):
      pltpu.sync_copy(srcp_hbm.at[pl.ds(s * _CPT + p * hc, hc)], sidx2)
      pltpu.sync_copy(dstp_hbm.at[pl.ds(s * _CPT + p * hc, hc)], didx2)
      pltpu.make_async_copy(q_hbm.at[sidx2.at[0]], r0, sem0).start()

      def body(i, carry):
        for b, rb, sem, rn, semn in ((0, r0, sem0, r1, sem1),
                                     (1, r1, sem1, r0, sem0)):
          j = 2 * i + b

          @pl.when(j < hc - 1)
          def _():
            pltpu.make_async_copy(q_hbm.at[sidx2.at[j + 1]], rn, semn).start()

          pltpu.make_async_copy(q_hbm.at[sidx2.at[j]], rb, sem).wait()
          pltpu.sync_copy(rb, acc.at[didx2.at[j]], add=True)
        return carry

      lax.fori_loop(0, hc // 2, body, 0)

  @pl.when(c == 0)
  def _():
    run_half(qa_hbm)

  @pl.when(c == 1)
  def _():
    run_half(qb_hbm)

  plsc.subcore_barrier()

  @pl.when(c == 0)
  def _():
    pltpu.sync_copy(acc.at[pl.ds(s * _RPT, _RPT)],
                    sa_hbm.at[pl.ds(s * _RPT, _RPT)])

  @pl.when(c == 1)
  def _():
    pltpu.sync_copy(acc.at[pl.ds(s * _RPT, _RPT)],
                    sb_hbm.at[pl.ds(s * _RPT, _RPT)])


@functools.partial(
    pl.kernel,
    out_type=(jax.ShapeDtypeStruct((80, _HW), _f32),
              jax.ShapeDtypeStruct((80, _HW), _f32)),
    mesh=_mesh,
    scratch_types=[
        pltpu.VMEM((8, _K), jnp.int32),
        pltpu.VMEM((_K, _HW), _f32),
        pltpu.VMEM_SHARED((80, _HW), _f32),
    ],
)
def _pool(ha_hbm, hb_hbm, batchp_hbm, z128_hbm, pa_hbm, pb_hbm,
          idx2, rows, accp):
  c = lax.axis_index("c")
  s = lax.axis_index("s")

  @pl.when(s == 0)
  def _():
    pltpu.sync_copy(z128_hbm.at[pl.ds(0, 80)], accp)

  @pl.when(s < 10)
  def _():
    pltpu.sync_copy(batchp_hbm.at[pl.ds(s * 8, 8)], idx2)
  plsc.subcore_barrier()

  def run_half(h_hbm):
    def body(j, carry):
      pltpu.sync_copy(h_hbm.at[pl.ds(s * 1024 + j * _K, _K)], rows)
      pltpu.sync_copy(rows, accp.at[idx2.at[j]], add=True)
      return carry
    lax.fori_loop(0, 8, body, 0)

  @pl.when(jnp.logical_and(c == 0, s < 10))
  def _():
    run_half(ha_hbm)

  @pl.when(jnp.logical_and(c == 1, s < 10))
  def _():
    run_half(hb_hbm)

  plsc.subcore_barrier()

  @pl.when(s == 0)
  def _():
    @pl.when(c == 0)
    def _():
      pltpu.sync_copy(accp, pa_hbm)

    @pl.when(c == 1)
    def _():
      pltpu.sync_copy(accp, pb_hbm)


def _dis_of(indeg_blk):
  return lax.rsqrt(indeg_blk[:, 0:1] + 1.0)


def _pre_body(x_ref, w_ref, indeg_ref, qa_ref, qb_ref):
  q = jnp.dot(x_ref[...], w_ref[...], preferred_element_type=_f32)
  q = q * _dis_of(indeg_ref[...])
  qa_ref[...] = q[:, :_HW]
  qb_ref[...] = q[:, _HW:]


def _row_spec(w):
  return pl.BlockSpec((_RB, w), lambda i: (i, 0))


def _full_spec(r, cx):
  return pl.BlockSpec((r, cx), lambda i: (0, 0))


_pre = pl.pallas_call(
    _pre_body,
    grid=(25,),
    in_specs=[_row_spec(_D), _full_spec(_D, _D), _row_spec(_HW)],
    out_specs=[_row_spec(_HW), _row_spec(_HW)],
    out_shape=[jax.ShapeDtypeStruct((_N, _HW), _f32)] * 2,
)


def _make_mid(extra_mix):
  def body(*refs):
    if extra_mix:
      (sa, sb, qa, qb, indeg, bc, wm, bm, wm2, bm2, wn, oa, ob) = refs
    else:
      (sa, sb, qa, qb, indeg, bc, wm, bm, wn, oa, ob) = refs
    dis = _dis_of(indeg[...])
    sfull = jnp.concatenate([sa[...] + qa[...], sb[...] + qb[...]], axis=1)
    h = jnp.maximum(dis * sfull + bc[...], 0.0)
    h = jnp.dot(h, wm[...], preferred_element_type=_f32) + bm[...]
    if extra_mix:
      h = jnp.dot(h, wm2[...], preferred_element_type=_f32) + bm2[...]
    q = jnp.dot(h, wn[...], preferred_element_type=_f32) * dis
    oa[...] = q[:, :_HW]
    ob[...] = q[:, _HW:]

  in_specs = ([_row_spec(_HW)] * 4 + [_row_spec(_HW), _full_spec(1, _D),
                                      _full_spec(_D, _D), _full_spec(1, _D)])
  if extra_mix:
    in_specs += [_full_spec(_D, _D), _full_spec(1, _D)]
  in_specs += [_full_spec(_D, _D)]
  return pl.pallas_call(
      body,
      grid=(25,),
      in_specs=in_specs,
      out_specs=[_row_spec(_HW), _row_spec(_HW)],
      out_shape=[jax.ShapeDtypeStruct((_N, _HW), _f32)] * 2,
  )


_mid_plain = _make_mid(False)
_mid_extra = _make_mid(True)


def _post_body(sa, sb, qa, qb, indeg, bc, wm2, bm2, oa, ob):
  dis = _dis_of(indeg[...])
  sfull = jnp.concatenate([sa[...] + qa[...], sb[...] + qb[...]], axis=1)
  t = dis * sfull + bc[...]
  h = jnp.maximum(jnp.dot(t, wm2[...], preferred_element_type=_f32)
                  + bm2[...], 0.0)
  oa[...] = h[:, :_HW]
  ob[...] = h[:, _HW:]


_post = pl.pallas_call(
    _post_body,
    grid=(25,),
    in_specs=[_row_spec(_HW)] * 4 + [_row_spec(_HW), _full_spec(1, _D),
                                     _full_spec(_D, _D), _full_spec(1, _D)],
    out_specs=[_row_spec(_HW), _row_spec(_HW)],
    out_shape=[jax.ShapeDtypeStruct((_NP, _HW), _f32)] * 2,
)


def _final_body(pa, pb, counts, wl, bl, out):
  sums = jnp.concatenate([pa[...][:_G], pb[...][:_G]], axis=1)
  cnt = counts[...][:_G, 0:1]
  pooled = sums / jnp.maximum(cnt, 1.0)
  out[...] = jnp.dot(pooled, wl[...], preferred_element_type=_f32) + bl[...]


_final = pl.pallas_call(
    _final_body,
    grid=(1,),
    in_specs=[_full_spec(80, _HW), _full_spec(80, _HW), _full_spec(80, _HW),
              _full_spec(_D, _HW), _full_spec(1, _HW)],
    out_specs=_full_spec(_G, _HW),
    out_shape=jax.ShapeDtypeStruct((_G, _HW), _f32),
)


def kernel(x, edge_index, batch, W1, b1, W2, b2, W3, b3, W4, b4,
           Wm, bm, Wm2, bm2, Wl, bl):
  i32 = jnp.int32
  src = edge_index[0].astype(i32)
  dst = edge_index[1].astype(i32)
  srcp = jnp.concatenate([src, jnp.zeros((_EP - _E,), i32)]).reshape(-1, _K)
  dstp = jnp.concatenate([dst, jnp.full((_EP - _E,), _N, i32)]).reshape(-1, _K)
  batchp = jnp.concatenate(
      [batch.astype(i32), jnp.full((_NP - _N,), _G, i32)]).reshape(-1, _K)
  z128 = jnp.zeros((_RPT, _HW), _f32)
  onesN = jnp.ones((_N, _HW), _f32)
  onesNP = jnp.ones((_NP, _HW), _f32)

  indeg, _ = _spmv(onesN, onesN, srcp, dstp, z128)
  counts, _ = _pool(onesNP, onesNP, batchp, z128)

  b1r, b2r, b3r, b4r = (b.reshape(1, -1) for b in (b1, b2, b3, b4))
  bmr, bm2r = bm.reshape(1, -1), bm2.reshape(1, -1)

  qa, qb = _pre(x, W1, indeg)
  sa, sb = _spmv(qa, qb, srcp, dstp, z128)
  qa, qb = _mid_plain(sa, sb, qa, qb, indeg, b1r, Wm, bmr, W2)
  sa, sb = _spmv(qa, qb, srcp, dstp, z128)
  qa, qb = _mid_extra(sa, sb, qa, qb, indeg, b2r, Wm, bmr, Wm2, bm2r, W3)
  sa, sb = _spmv(qa, qb, srcp, dstp, z128)
  qa, qb = _mid_plain(sa, sb, qa, qb, indeg, b3r, Wm, bmr, W4)
  sa, sb = _spmv(qa, qb, srcp, dstp, z128)
  qa, qb = _mid_plain(sa, sb, qa, qb, indeg, b4r, Wm, bmr, W4)
  sa, sb = _spmv(qa, qb, srcp, dstp, z128)
  ha, hb = _post(sa, sb, qa, qb, indeg, b4r, Wm2, bm2r)

  pa, pb = _pool(ha, hb, batchp, z128)

  wlp = jnp.pad(Wl, ((0, 0), (0, _HW - Wl.shape[1])))
  blp = jnp.pad(bl, (0, _HW - bl.shape[0])).reshape(1, -1)
  out = _final(pa, pb, counts, wlp, blp)
  return out[:, :bl.shape[0]]


# speedup vs baseline: 6.7028x; 1.2248x over previous
import functools

import jax
import jax.numpy as jnp
from jax import lax
from jax.experimental import pallas as pl
from jax.experimental.pallas import tpu as pltpu
from jax.experimental.pallas import tpu_sc as plsc

_N = 10000
_E = 160000
_D = 256
_G = 64
_HW = 128
_K = 128
_NP = 10240
_EP = 163840
_EPT = _EP // 16
_CPT = _EPT // _K
_RPT = _NP // 16
_RB = 400

_mesh = plsc.VectorSubcoreMesh(
    core_axis_name="c", subcore_axis_name="s", num_cores=2, num_subcores=16)

_f32 = jnp.float32


@functools.partial(
    pl.kernel,
    out_type=(jax.ShapeDtypeStruct((_NP, _HW), _f32),
              jax.ShapeDtypeStruct((_NP, _HW), _f32),
              jax.ShapeDtypeStruct((80, _HW), _f32)),
    mesh=_mesh,
    scratch_types=[
        pltpu.VMEM((_CPT // 2, _K), jnp.int32),
        pltpu.VMEM((8, _K), jnp.int32),
        pltpu.VMEM((_K, _HW), _f32),
        pltpu.VMEM_SHARED((_NP, _HW), _f32),
        pltpu.VMEM_SHARED((80, _HW), _f32),
    ],
)
def _hist(dstp_hbm, batchp_hbm, z128_hbm, o128_hbm, dega_hbm, degb_hbm,
          counts_hbm, didx2, bidx2, ones_v, accd, accc):
  c = lax.axis_index("c")
  s = lax.axis_index("s")
  hc = _CPT // 2
  pltpu.sync_copy(o128_hbm, ones_v)
  pltpu.sync_copy(z128_hbm, accd.at[pl.ds(s * _RPT, _RPT)])
  pltpu.sync_copy(dstp_hbm.at[pl.ds((c * 16 + s) * hc, hc)], didx2)

  @pl.when(jnp.logical_and(c == 1, s == 0))
  def _():
    pltpu.sync_copy(z128_hbm.at[pl.ds(0, 80)], accc)

  @pl.when(jnp.logical_and(c == 1, s < 10))
  def _():
    pltpu.sync_copy(batchp_hbm.at[pl.ds(s * 8, 8)], bidx2)

  plsc.subcore_barrier()

  def body(j, carry):
    pltpu.sync_copy(ones_v, accd.at[didx2.at[j]], add=True)
    return carry
  lax.fori_loop(0, hc, body, 0)

  @pl.when(jnp.logical_and(c == 1, s < 10))
  def _():
    def bbody(j, carry):
      pltpu.sync_copy(ones_v, accc.at[bidx2.at[j]], add=True)
      return carry
    lax.fori_loop(0, 8, bbody, 0)

  plsc.subcore_barrier()

  @pl.when(c == 0)
  def _():
    pltpu.sync_copy(accd.at[pl.ds(s * _RPT, _RPT)],
                    dega_hbm.at[pl.ds(s * _RPT, _RPT)])

  @pl.when(c == 1)
  def _():
    pltpu.sync_copy(accd.at[pl.ds(s * _RPT, _RPT)],
                    degb_hbm.at[pl.ds(s * _RPT, _RPT)])

  @pl.when(jnp.logical_and(c == 1, s == 0))
  def _():
    pltpu.sync_copy(accc, counts_hbm)


@functools.partial(
    pl.kernel,
    out_type=(jax.ShapeDtypeStruct((_NP, _HW), _f32),
              jax.ShapeDtypeStruct((_NP, _HW), _f32)),
    mesh=_mesh,
    scratch_types=[
        pltpu.VMEM((_CPT // 2, _K), jnp.int32),
        pltpu.VMEM((_CPT // 2, _K), jnp.int32),
        pltpu.VMEM((_K, _HW), _f32),
        pltpu.VMEM((_K, _HW), _f32),
        pltpu.VMEM_SHARED((_NP, _HW), _f32),
        pltpu.SemaphoreType.DMA,
        pltpu.SemaphoreType.DMA,
    ],
)
def _spmv(qa_hbm, qb_hbm, srcp_hbm, dstp_hbm, z128_hbm, sa_hbm, sb_hbm,
          sidx2, didx2, r0, r1, acc, sem0, sem1):
  c = lax.axis_index("c")
  s = lax.axis_index("s")
  hc = _CPT // 2
  pltpu.sync_copy(z128_hbm, acc.at[pl.ds(s * _RPT, _RPT)])
  plsc.subcore_barrier()

  def run_half(q_hbm):
    for p in (0, 1):
      pltpu.sync_copy(srcp_hbm.at[pl.ds(s * _CPT + p * hc, hc)], sidx2)
      pltpu.sync_copy(dstp_hbm.at[pl.ds(s * _CPT + p * hc, hc)], didx2)
      pltpu.make_async_copy(q_hbm.at[sidx2.at[0]], r0, sem0).start()

      def body(i, carry):
        for b, rb, sem, rn, semn in ((0, r0, sem0, r1, sem1),
                                     (1, r1, sem1, r0, sem0)):
          j = 2 * i + b

          @pl.when(j < hc - 1)
          def _():
            pltpu.make_async_copy(q_hbm.at[sidx2.at[j + 1]], rn, semn).start()

          pltpu.make_async_copy(q_hbm.at[sidx2.at[j]], rb, sem).wait()
          pltpu.sync_copy(rb, acc.at[didx2.at[j]], add=True)
        return carry

      lax.fori_loop(0, hc // 2, body, 0)

  @pl.when(c == 0)
  def _():
    run_half(qa_hbm)

  @pl.when(c == 1)
  def _():
    run_half(qb_hbm)

  plsc.subcore_barrier()

  @pl.when(c == 0)
  def _():
    pltpu.sync_copy(acc.at[pl.ds(s * _RPT, _RPT)],
                    sa_hbm.at[pl.ds(s * _RPT, _RPT)])

  @pl.when(c == 1)
  def _():
    pltpu.sync_copy(acc.at[pl.ds(s * _RPT, _RPT)],
                    sb_hbm.at[pl.ds(s * _RPT, _RPT)])


@functools.partial(
    pl.kernel,
    out_type=(jax.ShapeDtypeStruct((80, _HW), _f32),
              jax.ShapeDtypeStruct((80, _HW), _f32)),
    mesh=_mesh,
    scratch_types=[
        pltpu.VMEM((8, _K), jnp.int32),
        pltpu.VMEM((_K, _HW), _f32),
        pltpu.VMEM_SHARED((80, _HW), _f32),
    ],
)
def _pool(ha_hbm, hb_hbm, batchp_hbm, z128_hbm, pa_hbm, pb_hbm,
          idx2, rows, accp):
  c = lax.axis_index("c")
  s = lax.axis_index("s")

  @pl.when(s == 0)
  def _():
    pltpu.sync_copy(z128_hbm.at[pl.ds(0, 80)], accp)

  @pl.when(s < 10)
  def _():
    pltpu.sync_copy(batchp_hbm.at[pl.ds(s * 8, 8)], idx2)
  plsc.subcore_barrier()

  def run_half(h_hbm):
    def body(j, carry):
      pltpu.sync_copy(h_hbm.at[pl.ds(s * 1024 + j * _K, _K)], rows)
      pltpu.sync_copy(rows, accp.at[idx2.at[j]], add=True)
      return carry
    lax.fori_loop(0, 8, body, 0)

  @pl.when(jnp.logical_and(c == 0, s < 10))
  def _():
    run_half(ha_hbm)

  @pl.when(jnp.logical_and(c == 1, s < 10))
  def _():
    run_half(hb_hbm)

  plsc.subcore_barrier()

  @pl.when(s == 0)
  def _():
    @pl.when(c == 0)
    def _():
      pltpu.sync_copy(accp, pa_hbm)

    @pl.when(c == 1)
    def _():
      pltpu.sync_copy(accp, pb_hbm)


def _dis_of(indeg_blk):
  return lax.rsqrt(indeg_blk[:, 0:1] + 1.0)


def _sumdeg_body(a_ref, b_ref, o_ref):
  o_ref[...] = a_ref[...] + b_ref[...]


_sumdeg = pl.pallas_call(
    _sumdeg_body,
    grid=(16,),
    in_specs=[pl.BlockSpec((_RPT, _HW), lambda i: (i, 0))] * 2,
    out_specs=pl.BlockSpec((_RPT, _HW), lambda i: (i, 0)),
    out_shape=jax.ShapeDtypeStruct((_NP, _HW), _f32),
)


def _pre_body(x_ref, w_ref, indeg_ref, qa_ref, qb_ref):
  q = jnp.dot(x_ref[...], w_ref[...], preferred_element_type=_f32)
  q = q * _dis_of(indeg_ref[...])
  qa_ref[...] = q[:, :_HW]
  qb_ref[...] = q[:, _HW:]


def _row_spec(w):
  return pl.BlockSpec((_RB, w), lambda i: (i, 0))


def _full_spec(r, cx):
  return pl.BlockSpec((r, cx), lambda i: (0, 0))


_pre = pl.pallas_call(
    _pre_body,
    grid=(25,),
    in_specs=[_row_spec(_D), _full_spec(_D, _D), _row_spec(_HW)],
    out_specs=[_row_spec(_HW), _row_spec(_HW)],
    out_shape=[jax.ShapeDtypeStruct((_N, _HW), _f32)] * 2,
)


def _make_mid(extra_mix):
  def body(*refs):
    if extra_mix:
      (sa, sb, qa, qb, indeg, bc, wm, bm, wm2, bm2, wn, oa, ob) = refs
    else:
      (sa, sb, qa, qb, indeg, bc, wm, bm, wn, oa, ob) = refs
    dis = _dis_of(indeg[...])
    sfull = jnp.concatenate([sa[...] + qa[...], sb[...] + qb[...]], axis=1)
    h = jnp.maximum(dis * sfull + bc[...], 0.0)
    h = jnp.dot(h, wm[...], preferred_element_type=_f32) + bm[...]
    if extra_mix:
      h = jnp.dot(h, wm2[...], preferred_element_type=_f32) + bm2[...]
    q = jnp.dot(h, wn[...], preferred_element_type=_f32) * dis
    oa[...] = q[:, :_HW]
    ob[...] = q[:, _HW:]

  in_specs = ([_row_spec(_HW)] * 4 + [_row_spec(_HW), _full_spec(1, _D),
                                      _full_spec(_D, _D), _full_spec(1, _D)])
  if extra_mix:
    in_specs += [_full_spec(_D, _D), _full_spec(1, _D)]
  in_specs += [_full_spec(_D, _D)]
  return pl.pallas_call(
      body,
      grid=(25,),
      in_specs=in_specs,
      out_specs=[_row_spec(_HW), _row_spec(_HW)],
      out_shape=[jax.ShapeDtypeStruct((_N, _HW), _f32)] * 2,
  )


_mid_plain = _make_mid(False)
_mid_extra = _make_mid(True)


def _post_body(sa, sb, qa, qb, indeg, bc, wm2, bm2, oa, ob):
  dis = _dis_of(indeg[...])
  sfull = jnp.concatenate([sa[...] + qa[...], sb[...] + qb[...]], axis=1)
  t = dis * sfull + bc[...]
  h = jnp.maximum(jnp.dot(t, wm2[...], preferred_element_type=_f32)
                  + bm2[...], 0.0)
  oa[...] = h[:, :_HW]
  ob[...] = h[:, _HW:]


_post = pl.pallas_call(
    _post_body,
    grid=(25,),
    in_specs=[_row_spec(_HW)] * 4 + [_row_spec(_HW), _full_spec(1, _D),
                                     _full_spec(_D, _D), _full_spec(1, _D)],
    out_specs=[_row_spec(_HW), _row_spec(_HW)],
    out_shape=[jax.ShapeDtypeStruct((_NP, _HW), _f32)] * 2,
)


def _final_body(pa, pb, counts, wl, bl, out):
  sums = jnp.concatenate([pa[...][:_G], pb[...][:_G]], axis=1)
  cnt = counts[...][:_G, 0:1]
  pooled = sums / jnp.maximum(cnt, 1.0)
  out[...] = jnp.dot(pooled, wl[...], preferred_element_type=_f32) + bl[...]


_final = pl.pallas_call(
    _final_body,
    grid=(1,),
    in_specs=[_full_spec(80, _HW), _full_spec(80, _HW), _full_spec(80, _HW),
              _full_spec(_D, _HW), _full_spec(1, _HW)],
    out_specs=_full_spec(_G, _HW),
    out_shape=jax.ShapeDtypeStruct((_G, _HW), _f32),
)


def kernel(x, edge_index, batch, W1, b1, W2, b2, W3, b3, W4, b4,
           Wm, bm, Wm2, bm2, Wl, bl):
  i32 = jnp.int32
  src = edge_index[0].astype(i32)
  dst = edge_index[1].astype(i32)
  srcp = jnp.concatenate([src, jnp.zeros((_EP - _E,), i32)]).reshape(-1, _K)
  dstp = jnp.concatenate([dst, jnp.full((_EP - _E,), _N, i32)]).reshape(-1, _K)
  batchp = jnp.concatenate(
      [batch.astype(i32), jnp.full((_NP - _N,), _G, i32)]).reshape(-1, _K)
  z128 = jnp.zeros((_RPT, _HW), _f32)
  o128 = jnp.ones((_K, _HW), _f32)

  dega, degb, counts = _hist(dstp, batchp, z128, o128)
  indeg = _sumdeg(dega, degb)

  b1r, b2r, b3r, b4r = (b.reshape(1, -1) for b in (b1, b2, b3, b4))
  bmr, bm2r = bm.reshape(1, -1), bm2.reshape(1, -1)

  qa, qb = _pre(x, W1, indeg)
  sa, sb = _spmv(qa, qb, srcp, dstp, z128)
  qa, qb = _mid_plain(sa, sb, qa, qb, indeg, b1r, Wm, bmr, W2)
  sa, sb = _spmv(qa, qb, srcp, dstp, z128)
  qa, qb = _mid_extra(sa, sb, qa, qb, indeg, b2r, Wm, bmr, Wm2, bm2r, W3)
  sa, sb = _spmv(qa, qb, srcp, dstp, z128)
  qa, qb = _mid_plain(sa, sb, qa, qb, indeg, b3r, Wm, bmr, W4)
  sa, sb = _spmv(qa, qb, srcp, dstp, z128)
  qa, qb = _mid_plain(sa, sb, qa, qb, indeg, b4r, Wm, bmr, W4)
  sa, sb = _spmv(qa, qb, srcp, dstp, z128)
  ha, hb = _post(sa, sb, qa, qb, indeg, b4r, Wm2, bm2r)

  pa, pb = _pool(ha, hb, batchp, z128)

  wlp = jnp.pad(Wl, ((0, 0), (0, _HW - Wl.shape[1])))
  blp = jnp.pad(bl, (0, _HW - bl.shape[0])).reshape(1, -1)
  out = _final(pa, pb, counts, wlp, blp)
  return out[:, :bl.shape[0]]


# speedup vs baseline: 6.7029x; 1.0000x over previous
import functools

import jax
import jax.numpy as jnp
from jax import lax
from jax.experimental import pallas as pl
from jax.experimental.pallas import tpu as pltpu
from jax.experimental.pallas import tpu_sc as plsc

_N = 10000
_E = 160000
_D = 256
_G = 64
_HW = 128
_K = 128
_NP = 10240
_EP = 163840
_EPT = _EP // 16
_CPT = _EPT // _K
_RPT = _NP // 16
_RB = 400

_mesh = plsc.VectorSubcoreMesh(
    core_axis_name="c", subcore_axis_name="s", num_cores=2, num_subcores=16)

_f32 = jnp.float32


@functools.partial(
    pl.kernel,
    out_type=(jax.ShapeDtypeStruct((_NP, _HW), _f32),
              jax.ShapeDtypeStruct((_NP, _HW), _f32),
              jax.ShapeDtypeStruct((80, _HW), _f32)),
    mesh=_mesh,
    scratch_types=[
        pltpu.VMEM((_CPT // 2, _K), jnp.int32),
        pltpu.VMEM((8, _K), jnp.int32),
        pltpu.VMEM((_K, _HW), _f32),
        pltpu.VMEM_SHARED((_NP, _HW), _f32),
        pltpu.VMEM_SHARED((80, _HW), _f32),
    ],
)
def _hist(dstp_hbm, batchp_hbm, z128_hbm, o128_hbm, dega_hbm, degb_hbm,
          counts_hbm, didx2, bidx2, ones_v, accd, accc):
  c = lax.axis_index("c")
  s = lax.axis_index("s")
  hc = _CPT // 2
  pltpu.sync_copy(o128_hbm, ones_v)
  pltpu.sync_copy(z128_hbm, accd.at[pl.ds(s * _RPT, _RPT)])
  pltpu.sync_copy(dstp_hbm.at[pl.ds((c * 16 + s) * hc, hc)], didx2)

  @pl.when(jnp.logical_and(c == 1, s == 0))
  def _():
    pltpu.sync_copy(z128_hbm.at[pl.ds(0, 80)], accc)

  @pl.when(jnp.logical_and(c == 1, s < 10))
  def _():
    pltpu.sync_copy(batchp_hbm.at[pl.ds(s * 8, 8)], bidx2)

  plsc.subcore_barrier()

  def body(j, carry):
    pltpu.sync_copy(ones_v, accd.at[didx2.at[j]], add=True)
    return carry
  lax.fori_loop(0, hc, body, 0)

  @pl.when(jnp.logical_and(c == 1, s < 10))
  def _():
    def bbody(j, carry):
      pltpu.sync_copy(ones_v, accc.at[bidx2.at[j]], add=True)
      return carry
    lax.fori_loop(0, 8, bbody, 0)

  plsc.subcore_barrier()

  @pl.when(c == 0)
  def _():
    pltpu.sync_copy(accd.at[pl.ds(s * _RPT, _RPT)],
                    dega_hbm.at[pl.ds(s * _RPT, _RPT)])

  @pl.when(c == 1)
  def _():
    pltpu.sync_copy(accd.at[pl.ds(s * _RPT, _RPT)],
                    degb_hbm.at[pl.ds(s * _RPT, _RPT)])

  @pl.when(jnp.logical_and(c == 1, s == 0))
  def _():
    pltpu.sync_copy(accc, counts_hbm)


@functools.partial(
    pl.kernel,
    out_type=(jax.ShapeDtypeStruct((_NP, _HW), _f32),
              jax.ShapeDtypeStruct((_NP, _HW), _f32)),
    mesh=_mesh,
    scratch_types=[
        pltpu.VMEM((_CPT // 2, _K), jnp.int32),
        pltpu.VMEM((_CPT // 2, _K), jnp.int32),
        pltpu.VMEM((_K, _HW), _f32),
        pltpu.VMEM((_K, _HW), _f32),
        pltpu.VMEM_SHARED((_NP, _HW), _f32),
        pltpu.SemaphoreType.DMA,
        pltpu.SemaphoreType.DMA,
        pltpu.SemaphoreType.DMA,
        pltpu.SemaphoreType.DMA,
    ],
)
def _spmv(qa_hbm, qb_hbm, srcp_hbm, dstp_hbm, z128_hbm, sa_hbm, sb_hbm,
          sidx2, didx2, r0, r1, acc, g0, g1, t0, t1):
  c = lax.axis_index("c")
  s = lax.axis_index("s")
  hc = _CPT // 2
  pltpu.sync_copy(z128_hbm, acc.at[pl.ds(s * _RPT, _RPT)])
  plsc.subcore_barrier()

  rbuf = (r0, r1)
  gsem = (g0, g1)
  tsem = (t0, t1)

  def run_half(q_hbm):
    def gather(j, b):
      pltpu.make_async_copy(q_hbm.at[sidx2.at[j]], rbuf[b], gsem[b]).start()

    def scat_desc(j, b):
      return pltpu.async_copy(rbuf[b], acc.at[didx2.at[j]], tsem[b], add=True)

    def scat_wait(b):
      pltpu.make_async_copy(rbuf[b], acc.at[didx2.at[0]], tsem[b]).wait()

    for p in (0, 1):
      pltpu.sync_copy(srcp_hbm.at[pl.ds(s * _CPT + p * hc, hc)], sidx2)
      pltpu.sync_copy(dstp_hbm.at[pl.ds(s * _CPT + p * hc, hc)], didx2)
      gather(0, 0)

      def body(ii, carry):
        for b in (0, 1):
          j = 2 * ii + b
          jp = j + 1

          @pl.when(jp < hc)
          def _():
            @pl.when(jp >= 2)
            def _():
              scat_wait(1 - b)
            gather(jp, 1 - b)

          pltpu.make_async_copy(q_hbm.at[sidx2.at[j]], rbuf[b],
                                gsem[b]).wait()
          scat_desc(j, b)
        return carry

      lax.fori_loop(0, hc // 2, body, 0)
      scat_wait(0)
      scat_wait(1)

  @pl.when(c == 0)
  def _():
    run_half(qa_hbm)

  @pl.when(c == 1)
  def _():
    run_half(qb_hbm)

  plsc.subcore_barrier()

  @pl.when(c == 0)
  def _():
    pltpu.sync_copy(acc.at[pl.ds(s * _RPT, _RPT)],
                    sa_hbm.at[pl.ds(s * _RPT, _RPT)])

  @pl.when(c == 1)
  def _():
    pltpu.sync_copy(acc.at[pl.ds(s * _RPT, _RPT)],
                    sb_hbm.at[pl.ds(s * _RPT, _RPT)])


@functools.partial(
    pl.kernel,
    out_type=(jax.ShapeDtypeStruct((80, _HW), _f32),
              jax.ShapeDtypeStruct((80, _HW), _f32)),
    mesh=_mesh,
    scratch_types=[
        pltpu.VMEM((8, _K), jnp.int32),
        pltpu.VMEM((_K, _HW), _f32),
        pltpu.VMEM_SHARED((80, _HW), _f32),
    ],
)
def _pool(ha_hbm, hb_hbm, batchp_hbm, z128_hbm, pa_hbm, pb_hbm,
          idx2, rows, accp):
  c = lax.axis_index("c")
  s = lax.axis_index("s")

  @pl.when(s == 0)
  def _():
    pltpu.sync_copy(z128_hbm.at[pl.ds(0, 80)], accp)

  @pl.when(s < 10)
  def _():
    pltpu.sync_copy(batchp_hbm.at[pl.ds(s * 8, 8)], idx2)
  plsc.subcore_barrier()

  def run_half(h_hbm):
    def body(j, carry):
      pltpu.sync_copy(h_hbm.at[pl.ds(s * 1024 + j * _K, _K)], rows)
      pltpu.sync_copy(rows, accp.at[idx2.at[j]], add=True)
      return carry
    lax.fori_loop(0, 8, body, 0)

  @pl.when(jnp.logical_and(c == 0, s < 10))
  def _():
    run_half(ha_hbm)

  @pl.when(jnp.logical_and(c == 1, s < 10))
  def _():
    run_half(hb_hbm)

  plsc.subcore_barrier()

  @pl.when(s == 0)
  def _():
    @pl.when(c == 0)
    def _():
      pltpu.sync_copy(accp, pa_hbm)

    @pl.when(c == 1)
    def _():
      pltpu.sync_copy(accp, pb_hbm)


def _dis_of(indeg_blk):
  return lax.rsqrt(indeg_blk[:, 0:1] + 1.0)


def _sumdeg_body(a_ref, b_ref, o_ref):
  o_ref[...] = a_ref[...] + b_ref[...]


_sumdeg = pl.pallas_call(
    _sumdeg_body,
    grid=(16,),
    in_specs=[pl.BlockSpec((_RPT, _HW), lambda i: (i, 0))] * 2,
    out_specs=pl.BlockSpec((_RPT, _HW), lambda i: (i, 0)),
    out_shape=jax.ShapeDtypeStruct((_NP, _HW), _f32),
)


def _pre_body(x_ref, w_ref, indeg_ref, qa_ref, qb_ref):
  q = jnp.dot(x_ref[...], w_ref[...], preferred_element_type=_f32)
  q = q * _dis_of(indeg_ref[...])
  qa_ref[...] = q[:, :_HW]
  qb_ref[...] = q[:, _HW:]


def _row_spec(w):
  return pl.BlockSpec((_RB, w), lambda i: (i, 0))


def _full_spec(r, cx):
  return pl.BlockSpec((r, cx), lambda i: (0, 0))


_pre = pl.pallas_call(
    _pre_body,
    grid=(25,),
    in_specs=[_row_spec(_D), _full_spec(_D, _D), _row_spec(_HW)],
    out_specs=[_row_spec(_HW), _row_spec(_HW)],
    out_shape=[jax.ShapeDtypeStruct((_N, _HW), _f32)] * 2,
)


def _make_mid(extra_mix):
  def body(*refs):
    if extra_mix:
      (sa, sb, qa, qb, indeg, bc, wm, bm, wm2, bm2, wn, oa, ob) = refs
    else:
      (sa, sb, qa, qb, indeg, bc, wm, bm, wn, oa, ob) = refs
    dis = _dis_of(indeg[...])
    sfull = jnp.concatenate([sa[...] + qa[...], sb[...] + qb[...]], axis=1)
    h = jnp.maximum(dis * sfull + bc[...], 0.0)
    h = jnp.dot(h, wm[...], preferred_element_type=_f32) + bm[...]
    if extra_mix:
      h = jnp.dot(h, wm2[...], preferred_element_type=_f32) + bm2[...]
    q = jnp.dot(h, wn[...], preferred_element_type=_f32) * dis
    oa[...] = q[:, :_HW]
    ob[...] = q[:, _HW:]

  in_specs = ([_row_spec(_HW)] * 4 + [_row_spec(_HW), _full_spec(1, _D),
                                      _full_spec(_D, _D), _full_spec(1, _D)])
  if extra_mix:
    in_specs += [_full_spec(_D, _D), _full_spec(1, _D)]
  in_specs += [_full_spec(_D, _D)]
  return pl.pallas_call(
      body,
      grid=(25,),
      in_specs=in_specs,
      out_specs=[_row_spec(_HW), _row_spec(_HW)],
      out_shape=[jax.ShapeDtypeStruct((_N, _HW), _f32)] * 2,
  )


_mid_plain = _make_mid(False)
_mid_extra = _make_mid(True)


def _post_body(sa, sb, qa, qb, indeg, bc, wm2, bm2, oa, ob):
  dis = _dis_of(indeg[...])
  sfull = jnp.concatenate([sa[...] + qa[...], sb[...] + qb[...]], axis=1)
  t = dis * sfull + bc[...]
  h = jnp.maximum(jnp.dot(t, wm2[...], preferred_element_type=_f32)
                  + bm2[...], 0.0)
  oa[...] = h[:, :_HW]
  ob[...] = h[:, _HW:]


_post = pl.pallas_call(
    _post_body,
    grid=(25,),
    in_specs=[_row_spec(_HW)] * 4 + [_row_spec(_HW), _full_spec(1, _D),
                                     _full_spec(_D, _D), _full_spec(1, _D)],
    out_specs=[_row_spec(_HW), _row_spec(_HW)],
    out_shape=[jax.ShapeDtypeStruct((_NP, _HW), _f32)] * 2,
)


def _final_body(pa, pb, counts, wl, bl, out):
  sums = jnp.concatenate([pa[...][:_G], pb[...][:_G]], axis=1)
  cnt = counts[...][:_G, 0:1]
  pooled = sums / jnp.maximum(cnt, 1.0)
  out[...] = jnp.dot(pooled, wl[...], preferred_element_type=_f32) + bl[...]


_final = pl.pallas_call(
    _final_body,
    grid=(1,),
    in_specs=[_full_spec(80, _HW), _full_spec(80, _HW), _full_spec(80, _HW),
              _full_spec(_D, _HW), _full_spec(1, _HW)],
    out_specs=_full_spec(_G, _HW),
    out_shape=jax.ShapeDtypeStruct((_G, _HW), _f32),
)


def kernel(x, edge_index, batch, W1, b1, W2, b2, W3, b3, W4, b4,
           Wm, bm, Wm2, bm2, Wl, bl):
  i32 = jnp.int32
  src = edge_index[0].astype(i32)
  dst = edge_index[1].astype(i32)
  srcp = jnp.concatenate([src, jnp.zeros((_EP - _E,), i32)]).reshape(-1, _K)
  dstp = jnp.concatenate([dst, jnp.full((_EP - _E,), _N, i32)]).reshape(-1, _K)
  batchp = jnp.concatenate(
      [batch.astype(i32), jnp.full((_NP - _N,), _G, i32)]).reshape(-1, _K)
  z128 = jnp.zeros((_RPT, _HW), _f32)
  o128 = jnp.ones((_K, _HW), _f32)

  dega, degb, counts = _hist(dstp, batchp, z128, o128)
  indeg = _sumdeg(dega, degb)

  b1r, b2r, b3r, b4r = (b.reshape(1, -1) for b in (b1, b2, b3, b4))
  bmr, bm2r = bm.reshape(1, -1), bm2.reshape(1, -1)

  qa, qb = _pre(x, W1, indeg)
  sa, sb = _spmv(qa, qb, srcp, dstp, z128)
  qa, qb = _mid_plain(sa, sb, qa, qb, indeg, b1r, Wm, bmr, W2)
  sa, sb = _spmv(qa, qb, srcp, dstp, z128)
  qa, qb = _mid_extra(sa, sb, qa, qb, indeg, b2r, Wm, bmr, Wm2, bm2r, W3)
  sa, sb = _spmv(qa, qb, srcp, dstp, z128)
  qa, qb = _mid_plain(sa, sb, qa, qb, indeg, b3r, Wm, bmr, W4)
  sa, sb = _spmv(qa, qb, srcp, dstp, z128)
  qa, qb = _mid_plain(sa, sb, qa, qb, indeg, b4r, Wm, bmr, W4)
  sa, sb = _spmv(qa, qb, srcp, dstp, z128)
  ha, hb = _post(sa, sb, qa, qb, indeg, b4r, Wm2, bm2r)

  pa, pb = _pool(ha, hb, batchp, z128)

  wlp = jnp.pad(Wl, ((0, 0), (0, _HW - Wl.shape[1])))
  blp = jnp.pad(bl, (0, _HW - bl.shape[0])).reshape(1, -1)
  out = _final(pa, pb, counts, wlp, blp)
  return out[:, :bl.shape[0]]
